# baseline (speedup 1.0000x reference)
import jax
import jax.numpy as jnp
from jax import lax
from jax.experimental import pallas as pl
from jax.experimental.pallas import tpu as pltpu
from jax.experimental.pallas import tpu_sc as plsc

CLS_ID = 1
B = 16
L = 4096
LP1 = L + 1
NLANE = 16
NSTEP = L // NLANE
OUT_PAD = 33 * 128
NL_PAD = 128


def _body(values_hbm, lengths_hbm, out_hbm, nl_hbm, in_v, out_v, len_v, nl_v):
    c = lax.axis_index("c")
    s = lax.axis_index("s")
    wid = s * 2 + c

    @pl.when(wid < B)
    def _work():
        row = wid
        pltpu.sync_copy(values_hbm.at[row], in_v)
        pltpu.sync_copy(lengths_hbm, len_v)
        lane = lax.iota(jnp.int32, NLANE)
        len_vec = len_v[...]
        my_len = jnp.sum(jnp.where(lane == row, len_vec, 0))

        def step(k, carry):
            j = k * NLANE
            v = in_v[pl.ds(j, NLANE)]
            pos = lane + j
            out_v[pl.ds(j + 1, NLANE)] = jnp.where(pos < my_len, v, 0)
            return carry

        lax.fori_loop(0, NSTEP, step, 0)

        head = out_v[pl.ds(0, NLANE)]
        out_v[pl.ds(0, NLANE)] = jnp.where(lane == 0, CLS_ID, head)
        zero = jnp.zeros((NLANE,), jnp.int32)
        for k in range(LP1 // NLANE + 1, OUT_PAD // NLANE):
            out_v[pl.ds(k * NLANE, NLANE)] = zero
        tail = out_v[pl.ds(L + 1, NLANE)]
        out_v[pl.ds(L + 1, NLANE)] = jnp.where(lane == 0, tail, 0)
        pltpu.sync_copy(out_v, out_hbm.at[row])

        @pl.when(wid == 0)
        def _newlen():
            nlv = len_vec + 1
            for k in range(NL_PAD // NLANE):
                nl_v[pl.ds(k * NLANE, NLANE)] = nlv
            pltpu.sync_copy(nl_v, nl_hbm)


_mesh = plsc.VectorSubcoreMesh(core_axis_name="c", subcore_axis_name="s")

_prepend = pl.kernel(
    _body,
    out_type=[
        jax.ShapeDtypeStruct((B, OUT_PAD), jnp.int32),
        jax.ShapeDtypeStruct((NL_PAD,), jnp.int32),
    ],
    mesh=_mesh,
    compiler_params=pltpu.CompilerParams(needs_layout_passes=False),
    scratch_types=[
        pltpu.VMEM((L,), jnp.int32),
        pltpu.VMEM((OUT_PAD,), jnp.int32),
        pltpu.VMEM((NLANE,), jnp.int32),
        pltpu.VMEM((NL_PAD,), jnp.int32),
    ],
)


def kernel(values, lengths):
    out_pad, nl_pad = _prepend(values.astype(jnp.int32), lengths.astype(jnp.int32))
    out = out_pad[:, :LP1].astype(values.dtype)
    new_lengths = nl_pad[:B].astype(lengths.dtype)
    return out, new_lengths


# speedup vs baseline: 1.0991x; 1.0991x over previous
import jax
import jax.numpy as jnp
from jax import lax
from jax.experimental import pallas as pl
from jax.experimental.pallas import tpu as pltpu
from jax.experimental.pallas import tpu_sc as plsc

CLS_ID = 1
B = 16
L = 4096
LP1 = L + 1
NLANE = 16
OUT_PAD = 33 * 128


def _body(values_hbm, lengths_hbm, out_hbm, nl_hbm, in_v, out_v, len_v, nl_v):
    row = lax.axis_index("s")
    pltpu.sync_copy(values_hbm.at[row], in_v)
    pltpu.sync_copy(lengths_hbm, len_v)
    lane = lax.iota(jnp.int32, NLANE)
    len_vec = len_v[...]
    my_len = jnp.sum(jnp.where(lane == row, len_vec, 0))

    @plsc.parallel_loop(0, L, step=NLANE, unroll=8)
    def _shift(j):
        v = in_v[pl.ds(j, NLANE)]
        out_v[pl.ds(j + 1, NLANE)] = jnp.where(lane + j < my_len, v, 0)

    head = out_v[pl.ds(0, NLANE)]
    out_v[pl.ds(0, NLANE)] = jnp.where(lane == 0, CLS_ID, head)
    pltpu.sync_copy(out_v, out_hbm.at[row])

    @pl.when(row == 0)
    def _newlen():
        nl_v[...] = len_vec + 1
        pltpu.sync_copy(nl_v, nl_hbm)


_mesh = plsc.VectorSubcoreMesh(
    core_axis_name="c", subcore_axis_name="s", num_cores=1
)

_prepend = pl.kernel(
    _body,
    out_type=[
        jax.ShapeDtypeStruct((B, OUT_PAD), jnp.int32),
        jax.ShapeDtypeStruct((B,), jnp.int32),
    ],
    mesh=_mesh,
    compiler_params=pltpu.CompilerParams(
        needs_layout_passes=False, skip_device_barrier=True
    ),
    scratch_types=[
        pltpu.VMEM((L,), jnp.int32),
        pltpu.VMEM((OUT_PAD,), jnp.int32),
        pltpu.VMEM((NLANE,), jnp.int32),
        pltpu.VMEM((NLANE,), jnp.int32),
    ],
)


def kernel(values, lengths):
    out_pad, new_lengths = _prepend(
        values.astype(jnp.int32), lengths.astype(jnp.int32)
    )
    out = out_pad[:, :LP1].astype(values.dtype)
    return out, new_lengths.astype(lengths.dtype)
